# Initial kernel scaffold; baseline (speedup 1.0000x reference)
import jax
import jax.numpy as jnp
from jax.experimental import pallas as pl


def kernel(x, Wg, We):
    raise NotImplementedError("write your pallas kernel here")


# speedup vs baseline: 2.4462x; 2.4462x over previous
import functools

import jax
import jax.numpy as jnp
from jax import lax
from jax.experimental import pallas as pl
from jax.experimental.pallas import tpu as pltpu

E = 8
D = 1024
TOPK = 2


def _moe_dense_body(x_ref, wg_ref, we_ref, out_ref, comb_ref, acc_ref):
    e = pl.program_id(1)

    @pl.when(e == 0)
    def _gate():
        xblk = x_ref[...]
        wg = wg_ref[...]
        logits = lax.dot_general(
            xblk, wg, (((1,), (1,)), ((), ())),
            preferred_element_type=jnp.float32)
        z = logits - jnp.max(logits, axis=-1, keepdims=True)
        p = jnp.exp(z)
        p = p / jnp.sum(p, axis=-1, keepdims=True)
        idx = lax.broadcasted_iota(jnp.int32, p.shape, 1)
        m1 = jnp.max(p, axis=-1, keepdims=True)
        i1 = jnp.min(jnp.where(p == m1, idx, E), axis=-1, keepdims=True)
        sel1 = idx == i1
        pm = jnp.where(sel1, -1.0, p)
        m2 = jnp.max(pm, axis=-1, keepdims=True)
        i2 = jnp.min(jnp.where(pm == m2, idx, E), axis=-1, keepdims=True)
        sel2 = idx == i2
        denom = m1 + m2 + 1e-9
        comb_ref[...] = (jnp.where(sel1, m1 / denom, 0.0)
                         + jnp.where(sel2, m2 / denom, 0.0))

    comb = comb_ref[...]
    eidx = lax.broadcasted_iota(jnp.int32, comb.shape, 1)
    scale = jnp.sum(jnp.where(eidx == e, comb, 0.0), axis=-1, keepdims=True)
    y = lax.dot_general(
        x_ref[...], we_ref[0], (((1,), (1,)), ((), ())),
        preferred_element_type=jnp.float32)

    @pl.when(e == 0)
    def _init():
        acc_ref[...] = scale * y

    @pl.when(e > 0)
    def _accum():
        acc_ref[...] += scale * y

    @pl.when(e == E - 1)
    def _flush():
        out_ref[...] = acc_ref[...]


def kernel(x, Wg, We):
    orig_shape = x.shape
    xt = x.reshape(-1, orig_shape[-1])
    T = xt.shape[0]
    BM = 1024
    grid = (T // BM, E)
    out = pl.pallas_call(
        _moe_dense_body,
        grid=grid,
        in_specs=[
            pl.BlockSpec((BM, D), lambda t, e: (t, 0)),
            pl.BlockSpec((E, D), lambda t, e: (0, 0)),
            pl.BlockSpec((1, D, D), lambda t, e: (e, 0, 0)),
        ],
        out_specs=pl.BlockSpec((BM, D), lambda t, e: (t, 0)),
        out_shape=jax.ShapeDtypeStruct((T, D), jnp.float32),
        scratch_shapes=[
            pltpu.VMEM((BM, E), jnp.float32),
            pltpu.VMEM((BM, D), jnp.float32),
        ],
    )(xt, Wg, We)
    return out.reshape(orig_shape)


# speedup vs baseline: 2.5127x; 1.0272x over previous
import functools

import jax
import jax.numpy as jnp
from jax import lax
from jax.experimental import pallas as pl
from jax.experimental.pallas import tpu as pltpu

E = 8
D = 1024
TOPK = 2


def _moe_dense_body(x_ref, wg_ref, we_ref, out_ref, comb_ref):
    d = pl.program_id(0)
    e = pl.program_id(1)

    @pl.when((e == 0) & (d == 0))
    def _gate():
        xblk = x_ref[...]
        wg = wg_ref[...]
        logits = lax.dot_general(
            xblk, wg, (((1,), (1,)), ((), ())),
            preferred_element_type=jnp.float32)
        z = logits - jnp.max(logits, axis=-1, keepdims=True)
        p = jnp.exp(z)
        p = p / jnp.sum(p, axis=-1, keepdims=True)
        idx = lax.broadcasted_iota(jnp.int32, p.shape, 1)
        m1 = jnp.max(p, axis=-1, keepdims=True)
        i1 = jnp.min(jnp.where(p == m1, idx, E), axis=-1, keepdims=True)
        sel1 = idx == i1
        pm = jnp.where(sel1, -1.0, p)
        m2 = jnp.max(pm, axis=-1, keepdims=True)
        i2 = jnp.min(jnp.where(pm == m2, idx, E), axis=-1, keepdims=True)
        sel2 = idx == i2
        denom = m1 + m2 + 1e-9
        comb_ref[...] = (jnp.where(sel1, m1 / denom, 0.0)
                         + jnp.where(sel2, m2 / denom, 0.0))

    comb = comb_ref[...]
    eidx = lax.broadcasted_iota(jnp.int32, comb.shape, 1)
    scale = jnp.sum(jnp.where(eidx == e, comb, 0.0), axis=-1, keepdims=True)
    y = lax.dot_general(
        x_ref[...], we_ref[0], (((1,), (1,)), ((), ())),
        preferred_element_type=jnp.float32)

    @pl.when(e == 0)
    def _init():
        out_ref[...] = scale * y

    @pl.when(e > 0)
    def _accum():
        out_ref[...] += scale * y


def kernel(x, Wg, We):
    orig_shape = x.shape
    xt = x.reshape(-1, orig_shape[-1])
    T = xt.shape[0]
    BM = 4096
    BD = 512
    grid = (D // BD, E)
    out = pl.pallas_call(
        _moe_dense_body,
        grid=grid,
        in_specs=[
            pl.BlockSpec((BM, D), lambda d, e: (0, 0)),
            pl.BlockSpec((E, D), lambda d, e: (0, 0)),
            pl.BlockSpec((1, BD, D), lambda d, e: (e, d, 0)),
        ],
        out_specs=pl.BlockSpec((BM, BD), lambda d, e: (0, d)),
        out_shape=jax.ShapeDtypeStruct((T, D), jnp.float32),
        scratch_shapes=[
            pltpu.VMEM((BM, E), jnp.float32),
        ],
    )(xt, Wg, We)
    return out.reshape(orig_shape)
